# baseline (device time: 262539 ns/iter reference)
import functools

import jax
import jax.numpy as jnp
from jax import lax
from jax.experimental import pallas as pl
from jax.experimental.pallas import tpu as pltpu

T = 1024
D = 2048
V_LOCAL = 16384
CHUNK = 2048
N_CHUNKS = V_LOCAL // CHUNK


def _stats_body(x_ref, w_ref, s_out, x_bf):
    j = pl.program_id(0)

    @pl.when(j == 0)
    def _():
        x_bf[...] = x_ref[...].astype(jnp.bfloat16)
        s_out[...] = jnp.zeros((T, 1), jnp.float32)

    w_bf = w_ref[...].astype(jnp.bfloat16)
    logits = lax.dot_general(
        x_bf[...], w_bf, (((1,), (0,)), ((), ())),
        preferred_element_type=jnp.float32,
    )
    s_out[...] += jnp.sum(jnp.exp(logits), axis=1, keepdims=True)


def _combine_body(s_ref, ll_ref, out_ref, rs, rll, send_sems, recv_sems):
    my_x = lax.axis_index("x")
    my_y = lax.axis_index("y")
    my_z = lax.axis_index("z")
    nbr = (my_x, 1 - my_y, my_z)

    barrier = pltpu.get_barrier_semaphore()
    pl.semaphore_signal(barrier, inc=1, device_id=nbr,
                        device_id_type=pl.DeviceIdType.MESH)
    pl.semaphore_wait(barrier, 1)

    copies = []
    for k, (src, dst) in enumerate(((s_ref, rs), (ll_ref, rll))):
        c = pltpu.make_async_remote_copy(
            src_ref=src, dst_ref=dst,
            send_sem=send_sems.at[k], recv_sem=recv_sems.at[k],
            device_id=nbr, device_id_type=pl.DeviceIdType.MESH,
        )
        c.start()
        copies.append(c)
    for c in copies:
        c.wait()

    out_ref[...] = jnp.log(s_ref[...] + rs[...]) - (ll_ref[...] + rll[...])

    @functools.partial(pl.run_scoped, sem2=pltpu.SemaphoreType.REGULAR)
    def _(sem2):
        pl.semaphore_signal(sem2, inc=1, device_id=nbr,
                            device_id_type=pl.DeviceIdType.MESH)
        pl.semaphore_wait(sem2, 1)


def kernel(x, W, labels):
    s = pl.pallas_call(
        _stats_body,
        grid=(N_CHUNKS,),
        in_specs=[
            pl.BlockSpec((T, D), lambda j: (0, 0)),
            pl.BlockSpec((D, CHUNK), lambda j: (0, j)),
        ],
        out_specs=pl.BlockSpec((T, 1), lambda j: (0, 0)),
        out_shape=jax.ShapeDtypeStruct((T, 1), jnp.float32),
        scratch_shapes=[pltpu.VMEM((T, D), jnp.bfloat16)],
        compiler_params=pltpu.CompilerParams(
            dimension_semantics=("arbitrary",),
            vmem_limit_bytes=100 * 1024 * 1024,
        ),
    )(x, W)

    my_y = lax.axis_index("y")
    idx = labels - my_y * V_LOCAL
    valid = (idx >= 0) & (idx < V_LOCAL)
    w_cols = jnp.take(W, jnp.clip(idx, 0, V_LOCAL - 1), axis=1)
    ll = jnp.where(valid, jnp.sum(x * w_cols.T, axis=1), 0.0)
    ll = ll.astype(jnp.float32).reshape(T, 1)

    nll = pl.pallas_call(
        _combine_body,
        in_specs=[pl.BlockSpec(memory_space=pltpu.VMEM)] * 2,
        out_specs=pl.BlockSpec(memory_space=pltpu.VMEM),
        out_shape=jax.ShapeDtypeStruct((T, 1), jnp.float32),
        scratch_shapes=[
            pltpu.VMEM((T, 1), jnp.float32),
            pltpu.VMEM((T, 1), jnp.float32),
            pltpu.SemaphoreType.DMA((2,)),
            pltpu.SemaphoreType.DMA((2,)),
        ],
        compiler_params=pltpu.CompilerParams(collective_id=0),
    )(s, ll)

    return nll.reshape(T)


# device time: 106428 ns/iter; 2.4668x vs baseline; 2.4668x over previous
import functools

import jax
import jax.numpy as jnp
from jax import lax
from jax.experimental import pallas as pl
from jax.experimental.pallas import tpu as pltpu

T = 1024
D = 2048
V_LOCAL = 16384
CHUNK = 2048
N_CHUNKS = V_LOCAL // CHUNK


def _stats_body(x_ref, w_ref, l_ref, s_out, lle_out):
    j = pl.program_id(0)

    @pl.when(j == 0)
    def _():
        s_out[...] = jnp.zeros((T, 1), jnp.float32)
        lle_out[...] = jnp.zeros((T, 1), jnp.float32)

    logits = lax.dot_general(
        x_ref[...], w_ref[...], (((1,), (0,)), ((), ())),
        preferred_element_type=jnp.float32,
        precision=lax.Precision.DEFAULT,
    )
    e = jnp.exp(logits)
    s_out[...] += jnp.sum(e, axis=1, keepdims=True)

    my_y = lax.axis_index("y")
    lbl = l_ref[...] - (my_y * V_LOCAL + j * CHUNK)
    cols = lax.broadcasted_iota(jnp.int32, (T, CHUNK), 1)
    lle_out[...] += jnp.sum(
        jnp.where(cols == lbl, e, 0.0), axis=1, keepdims=True
    )


def _combine_body(s_ref, lle_ref, out_ref, rs, rlle, send_sems, recv_sems):
    my_x = lax.axis_index("x")
    my_y = lax.axis_index("y")
    my_z = lax.axis_index("z")
    nbr = (my_x, 1 - my_y, my_z)

    barrier = pltpu.get_barrier_semaphore()
    pl.semaphore_signal(barrier, inc=1, device_id=nbr,
                        device_id_type=pl.DeviceIdType.MESH)
    pl.semaphore_wait(barrier, 1)

    copies = []
    for k, (src, dst) in enumerate(((s_ref, rs), (lle_ref, rlle))):
        c = pltpu.make_async_remote_copy(
            src_ref=src, dst_ref=dst,
            send_sem=send_sems.at[k], recv_sem=recv_sems.at[k],
            device_id=nbr, device_id_type=pl.DeviceIdType.MESH,
        )
        c.start()
        copies.append(c)
    for c in copies:
        c.wait()

    out_ref[...] = jnp.log(s_ref[...] + rs[...]) - jnp.log(
        lle_ref[...] + rlle[...]
    )

    @functools.partial(pl.run_scoped, sem2=pltpu.SemaphoreType.REGULAR)
    def _(sem2):
        pl.semaphore_signal(sem2, inc=1, device_id=nbr,
                            device_id_type=pl.DeviceIdType.MESH)
        pl.semaphore_wait(sem2, 1)


def kernel(x, W, labels):
    labels2d = labels.reshape(T, 1)

    s, lle = pl.pallas_call(
        _stats_body,
        grid=(N_CHUNKS,),
        in_specs=[
            pl.BlockSpec((T, D), lambda j: (0, 0)),
            pl.BlockSpec((D, CHUNK), lambda j: (0, j)),
            pl.BlockSpec((T, 1), lambda j: (0, 0)),
        ],
        out_specs=[
            pl.BlockSpec((T, 1), lambda j: (0, 0)),
            pl.BlockSpec((T, 1), lambda j: (0, 0)),
        ],
        out_shape=[
            jax.ShapeDtypeStruct((T, 1), jnp.float32),
            jax.ShapeDtypeStruct((T, 1), jnp.float32),
        ],
        compiler_params=pltpu.CompilerParams(
            dimension_semantics=("arbitrary",),
            vmem_limit_bytes=100 * 1024 * 1024,
        ),
    )(x, W, labels2d)

    nll = pl.pallas_call(
        _combine_body,
        in_specs=[pl.BlockSpec(memory_space=pltpu.VMEM)] * 2,
        out_specs=pl.BlockSpec(memory_space=pltpu.VMEM),
        out_shape=jax.ShapeDtypeStruct((T, 1), jnp.float32),
        scratch_shapes=[
            pltpu.VMEM((T, 1), jnp.float32),
            pltpu.VMEM((T, 1), jnp.float32),
            pltpu.SemaphoreType.DMA((2,)),
            pltpu.SemaphoreType.DMA((2,)),
        ],
        compiler_params=pltpu.CompilerParams(collective_id=0),
    )(s, lle)

    return nll.reshape(T)


# device time: 35632 ns/iter; 7.3681x vs baseline; 2.9869x over previous
import functools

import jax
import jax.numpy as jnp
from jax import lax
from jax.experimental import pallas as pl
from jax.experimental.pallas import tpu as pltpu

T = 1024
D = 2048
V_LOCAL = 16384
V_SUB = 2048
HALF = V_SUB // 2


def _stats_body(x_ref, w_ref, l_ref, s_out, lle_out, wv, dma_sems):
    my_x = lax.axis_index("x")
    my_y = lax.axis_index("y")
    my_z = lax.axis_index("z")
    r = my_x * 4 + my_z
    c0 = r * V_SUB

    cps = []
    for h in range(2):
        cp = pltpu.make_async_copy(
            w_ref.at[:, pl.ds(c0 + h * HALF, HALF)],
            wv.at[:, pl.ds(h * HALF, HALF)],
            dma_sems.at[h],
        )
        cp.start()
        cps.append(cp)

    col_base = my_y * V_LOCAL + c0
    s = jnp.zeros((T, 1), jnp.float32)
    lle = jnp.zeros((T, 1), jnp.float32)
    for h in range(2):
        cps[h].wait()
        logits = lax.dot_general(
            x_ref[...], wv[:, h * HALF:(h + 1) * HALF],
            (((1,), (0,)), ((), ())),
            preferred_element_type=jnp.float32,
            precision=lax.Precision.DEFAULT,
        )
        e = jnp.exp(logits)
        s += jnp.sum(e, axis=1, keepdims=True)
        cols = lax.broadcasted_iota(jnp.int32, (T, HALF), 1) + (
            col_base + h * HALF
        )
        lle += jnp.sum(
            jnp.where(cols == l_ref[...], e, 0.0), axis=1, keepdims=True
        )
    s_out[...] = s
    lle_out[...] = lle


def _allreduce_body(st_ref, out_ref, acc, zrecv, xyrecv,
                    zs_sems, zr_sems, xs_sems, xr_sems):
    my_x = lax.axis_index("x")
    my_y = lax.axis_index("y")
    my_z = lax.axis_index("z")
    q = my_x * 2 + my_y

    def z_peer(d):
        return (my_x, my_y, (my_z + d) % 4)

    def xy_peer(d):
        pq = (q + d) % 4
        return (pq // 2, pq % 2, my_z)

    peers = [z_peer(d) for d in range(1, 4)] + [xy_peer(d) for d in range(1, 4)]

    barrier = pltpu.get_barrier_semaphore()
    for p in peers:
        pl.semaphore_signal(barrier, inc=1, device_id=p,
                            device_id_type=pl.DeviceIdType.MESH)
    pl.semaphore_wait(barrier, len(peers))

    zcopies = []
    for d in range(1, 4):
        c = pltpu.make_async_remote_copy(
            src_ref=st_ref, dst_ref=zrecv.at[d - 1],
            send_sem=zs_sems.at[d - 1], recv_sem=zr_sems.at[d - 1],
            device_id=z_peer(d), device_id_type=pl.DeviceIdType.MESH,
        )
        c.start()
        zcopies.append(c)
    for c in zcopies:
        c.wait()
    acc[...] = st_ref[...] + zrecv[0] + zrecv[1] + zrecv[2]

    xycopies = []
    for d in range(1, 4):
        c = pltpu.make_async_remote_copy(
            src_ref=acc, dst_ref=xyrecv.at[d - 1],
            send_sem=xs_sems.at[d - 1], recv_sem=xr_sems.at[d - 1],
            device_id=xy_peer(d), device_id_type=pl.DeviceIdType.MESH,
        )
        c.start()
        xycopies.append(c)
    for c in xycopies:
        c.wait()
    total = acc[...] + xyrecv[0] + xyrecv[1] + xyrecv[2]

    out_ref[...] = jnp.log(total[0:8, :]) - jnp.log(total[8:16, :])

    @functools.partial(pl.run_scoped, sem2=pltpu.SemaphoreType.REGULAR)
    def _(sem2):
        for p in peers:
            pl.semaphore_signal(sem2, inc=1, device_id=p,
                                device_id_type=pl.DeviceIdType.MESH)
        pl.semaphore_wait(sem2, len(peers))


def kernel(x, W, labels):
    labels2d = labels.reshape(T, 1)

    s, lle = pl.pallas_call(
        _stats_body,
        in_specs=[
            pl.BlockSpec(memory_space=pltpu.VMEM),
            pl.BlockSpec(memory_space=pl.ANY),
            pl.BlockSpec(memory_space=pltpu.VMEM),
        ],
        out_specs=[
            pl.BlockSpec(memory_space=pltpu.VMEM),
            pl.BlockSpec(memory_space=pltpu.VMEM),
        ],
        out_shape=[
            jax.ShapeDtypeStruct((T, 1), jnp.float32),
            jax.ShapeDtypeStruct((T, 1), jnp.float32),
        ],
        scratch_shapes=[
            pltpu.VMEM((D, V_SUB), jnp.float32),
            pltpu.SemaphoreType.DMA((2,)),
        ],
        compiler_params=pltpu.CompilerParams(
            vmem_limit_bytes=100 * 1024 * 1024,
        ),
    )(x, W, labels2d)

    st = jnp.concatenate([s.reshape(8, 128), lle.reshape(8, 128)], axis=0)

    total = pl.pallas_call(
        _allreduce_body,
        in_specs=[pl.BlockSpec(memory_space=pltpu.VMEM)],
        out_specs=pl.BlockSpec(memory_space=pltpu.VMEM),
        out_shape=jax.ShapeDtypeStruct((8, 128), jnp.float32),
        scratch_shapes=[
            pltpu.VMEM((16, 128), jnp.float32),
            pltpu.VMEM((3, 16, 128), jnp.float32),
            pltpu.VMEM((3, 16, 128), jnp.float32),
            pltpu.SemaphoreType.DMA((3,)),
            pltpu.SemaphoreType.DMA((3,)),
            pltpu.SemaphoreType.DMA((3,)),
            pltpu.SemaphoreType.DMA((3,)),
        ],
        compiler_params=pltpu.CompilerParams(collective_id=0),
    )(st)

    return total.reshape(T)


# device time: 31248 ns/iter; 8.4018x vs baseline; 1.1403x over previous
import functools

import jax
import jax.numpy as jnp
from jax import lax
from jax.experimental import pallas as pl
from jax.experimental.pallas import tpu as pltpu

T = 1024
D = 2048
V_LOCAL = 16384
V_SUB = 2048
N_CHUNKS = 4
CW = V_SUB // N_CHUNKS


def _body(x_ref, w_ref, l_ref, out_ref, wv, stbuf, acc, zrecv, xyrecv,
          dma_sems, zs_sems, zr_sems, xs_sems, xr_sems):
    my_x = lax.axis_index("x")
    my_y = lax.axis_index("y")
    my_z = lax.axis_index("z")
    r = my_x * 4 + my_z
    c0 = r * V_SUB
    q = my_x * 2 + my_y

    def z_peer(d):
        return (my_x, my_y, (my_z + d) % 4)

    def xy_peer(d):
        pq = (q + d) % 4
        return (pq // 2, pq % 2, my_z)

    peers = [z_peer(d) for d in range(1, 4)] + [xy_peer(d) for d in range(1, 4)]

    barrier = pltpu.get_barrier_semaphore()
    for p in peers:
        pl.semaphore_signal(barrier, inc=1, device_id=p,
                            device_id_type=pl.DeviceIdType.MESH)

    cps = []
    for h in range(N_CHUNKS):
        cp = pltpu.make_async_copy(
            w_ref.at[:, pl.ds(c0 + h * CW, CW)],
            wv.at[:, pl.ds(h * CW, CW)],
            dma_sems.at[h],
        )
        cp.start()
        cps.append(cp)

    col_base = my_y * V_LOCAL + c0
    ones_row = jnp.ones((1, CW), jnp.float32)
    red_dims = (((1,), (1,)), ((), ()))
    s_row = jnp.zeros((1, T), jnp.float32)
    lle_row = jnp.zeros((1, T), jnp.float32)
    for h in range(N_CHUNKS):
        cps[h].wait()
        logits = lax.dot_general(
            x_ref[...], wv[:, h * CW:(h + 1) * CW],
            (((1,), (0,)), ((), ())),
            preferred_element_type=jnp.float32,
            precision=lax.Precision.DEFAULT,
        )
        e = jnp.exp(logits)
        cols = lax.broadcasted_iota(jnp.int32, (T, CW), 1) + (
            col_base + h * CW
        )
        masked = jnp.where(cols == l_ref[...], e, 0.0)
        s_row += lax.dot_general(
            ones_row, e, red_dims,
            preferred_element_type=jnp.float32,
            precision=lax.Precision.DEFAULT,
        )
        lle_row += lax.dot_general(
            ones_row, masked, red_dims,
            preferred_element_type=jnp.float32,
            precision=lax.Precision.DEFAULT,
        )
    stbuf[0:1, :] = s_row
    stbuf[1:2, :] = lle_row

    pl.semaphore_wait(barrier, len(peers))

    zcopies = []
    for d in range(1, 4):
        c = pltpu.make_async_remote_copy(
            src_ref=stbuf, dst_ref=zrecv.at[d - 1],
            send_sem=zs_sems.at[d - 1], recv_sem=zr_sems.at[d - 1],
            device_id=z_peer(d), device_id_type=pl.DeviceIdType.MESH,
        )
        c.start()
        zcopies.append(c)
    for c in zcopies:
        c.wait()
    acc[...] = stbuf[...] + zrecv[0] + zrecv[1] + zrecv[2]

    xycopies = []
    for d in range(1, 4):
        c = pltpu.make_async_remote_copy(
            src_ref=acc, dst_ref=xyrecv.at[d - 1],
            send_sem=xs_sems.at[d - 1], recv_sem=xr_sems.at[d - 1],
            device_id=xy_peer(d), device_id_type=pl.DeviceIdType.MESH,
        )
        c.start()
        xycopies.append(c)
    for c in xycopies:
        c.wait()
    total = acc[...] + xyrecv[0] + xyrecv[1] + xyrecv[2]

    out_ref[...] = jnp.log(total[0:1, :]) - jnp.log(total[1:2, :])

    @functools.partial(pl.run_scoped, sem2=pltpu.SemaphoreType.REGULAR)
    def _(sem2):
        for p in peers:
            pl.semaphore_signal(sem2, inc=1, device_id=p,
                                device_id_type=pl.DeviceIdType.MESH)
        pl.semaphore_wait(sem2, len(peers))


def kernel(x, W, labels):
    labels2d = labels.reshape(T, 1)

    nll = pl.pallas_call(
        _body,
        in_specs=[
            pl.BlockSpec(memory_space=pltpu.VMEM),
            pl.BlockSpec(memory_space=pl.ANY),
            pl.BlockSpec(memory_space=pltpu.VMEM),
        ],
        out_specs=pl.BlockSpec(memory_space=pltpu.VMEM),
        out_shape=jax.ShapeDtypeStruct((1, T), jnp.float32),
        scratch_shapes=[
            pltpu.VMEM((D, V_SUB), jnp.float32),
            pltpu.VMEM((2, T), jnp.float32),
            pltpu.VMEM((2, T), jnp.float32),
            pltpu.VMEM((3, 2, T), jnp.float32),
            pltpu.VMEM((3, 2, T), jnp.float32),
            pltpu.SemaphoreType.DMA((N_CHUNKS,)),
            pltpu.SemaphoreType.DMA((3,)),
            pltpu.SemaphoreType.DMA((3,)),
            pltpu.SemaphoreType.DMA((3,)),
            pltpu.SemaphoreType.DMA((3,)),
        ],
        compiler_params=pltpu.CompilerParams(
            collective_id=0,
            vmem_limit_bytes=100 * 1024 * 1024,
        ),
    )(x, W, labels2d)

    return nll.reshape(T)
